# baseline (speedup 1.0000x reference)
import functools

import jax
import jax.numpy as jnp
from jax.experimental import pallas as pl
from jax.experimental.pallas import tpu as pltpu

_CH = 368
_TAIL = 80
_NBUF = 3


def _chunk_sizes(n):
    sizes = []
    rem = n - _TAIL
    while rem >= _CH:
        sizes.append(_CH)
        rem -= _CH
    if rem:
        sizes.append(rem)
    sizes.append(_TAIL)
    return sizes


def _gcn_kernel(x_ref, w_ref, b_ref, adj_hbm, out_ref, support_ref, bufs, sems):
    n = adj_hbm.shape[0]
    sizes = _chunk_sizes(n)
    offs = [0]
    for s in sizes:
        offs.append(offs[-1] + s)

    def start(i):
        pltpu.make_async_copy(
            adj_hbm.at[pl.ds(offs[i], sizes[i]), :],
            bufs.at[i % _NBUF, pl.ds(0, sizes[i]), :],
            sems.at[i % _NBUF],
        ).start()

    def wait(i):
        pltpu.make_async_copy(
            adj_hbm.at[pl.ds(offs[i], sizes[i]), :],
            bufs.at[i % _NBUF, pl.ds(0, sizes[i]), :],
            sems.at[i % _NBUF],
        ).wait()

    nch = len(sizes)
    for i in range(min(_NBUF, nch)):
        start(i)

    support_ref[...] = jnp.dot(
        x_ref[...], w_ref[...], preferred_element_type=jnp.float32
    )

    for i in range(nch):
        wait(i)
        blk = bufs[i % _NBUF, pl.ds(0, sizes[i]), :]
        o = (
            jnp.dot(blk, support_ref[...], preferred_element_type=jnp.float32)
            + b_ref[...]
        )
        m = jnp.max(o, axis=1, keepdims=True)
        e = o - m
        out_ref[pl.ds(offs[i], sizes[i]), :] = e - jnp.log(
            jnp.sum(jnp.exp(e), axis=1, keepdims=True)
        )
        if i + _NBUF < nch:
            start(i + _NBUF)


@jax.jit
def kernel(x, adj, W, b):
    n, nfeat = x.shape
    nclass = W.shape[1]
    b2 = b.reshape(1, nclass)
    return pl.pallas_call(
        _gcn_kernel,
        in_specs=[
            pl.BlockSpec(memory_space=pltpu.MemorySpace.VMEM),
            pl.BlockSpec(memory_space=pltpu.MemorySpace.VMEM),
            pl.BlockSpec(memory_space=pltpu.MemorySpace.VMEM),
            pl.BlockSpec(memory_space=pl.ANY),
        ],
        out_specs=pl.BlockSpec(memory_space=pltpu.MemorySpace.VMEM),
        out_shape=jax.ShapeDtypeStruct((n, nclass), jnp.float32),
        scratch_shapes=[
            pltpu.VMEM((n, nclass), jnp.float32),
            pltpu.VMEM((_NBUF, _CH, n), jnp.float32),
            pltpu.SemaphoreType.DMA((_NBUF,)),
        ],
    )(x, W, b2, adj)


# speedup vs baseline: 1.0012x; 1.0012x over previous
import functools

import jax
import jax.numpy as jnp
from jax.experimental import pallas as pl
from jax.experimental.pallas import tpu as pltpu

_CH = 368
_TAIL = 80
_NBUF = 3


def _chunk_sizes(n):
    sizes = []
    rem = n - _TAIL
    while rem >= _CH:
        sizes.append(_CH)
        rem -= _CH
    if rem:
        sizes.append(rem)
    sizes.append(_TAIL)
    return sizes


def _gcn_kernel(x_ref, w_ref, b_ref, adj_hbm, out_ref, support_ref, bufs, sems):
    n = adj_hbm.shape[0]
    sizes = _chunk_sizes(n)
    offs = [0]
    for s in sizes:
        offs.append(offs[-1] + s)

    def _copies(i):
        half = sizes[i] // 2
        return [
            pltpu.make_async_copy(
                adj_hbm.at[pl.ds(offs[i] + h * half, half), :],
                bufs.at[i % _NBUF, pl.ds(h * half, half), :],
                sems.at[i % _NBUF, h],
            )
            for h in range(2)
        ]

    def start(i):
        for c in _copies(i):
            c.start()

    def wait(i):
        for c in _copies(i):
            c.wait()

    nch = len(sizes)
    for i in range(min(_NBUF, nch)):
        start(i)

    support_ref[...] = jnp.dot(
        x_ref[...], w_ref[...], preferred_element_type=jnp.float32
    )

    for i in range(nch):
        wait(i)
        blk = bufs[i % _NBUF, pl.ds(0, sizes[i]), :]
        o = (
            jnp.dot(blk, support_ref[...], preferred_element_type=jnp.float32)
            + b_ref[...]
        )
        m = jnp.max(o, axis=1, keepdims=True)
        e = o - m
        out_ref[pl.ds(offs[i], sizes[i]), :] = e - jnp.log(
            jnp.sum(jnp.exp(e), axis=1, keepdims=True)
        )
        if i + _NBUF < nch:
            start(i + _NBUF)


@jax.jit
def kernel(x, adj, W, b):
    n, nfeat = x.shape
    nclass = W.shape[1]
    b2 = b.reshape(1, nclass)
    return pl.pallas_call(
        _gcn_kernel,
        in_specs=[
            pl.BlockSpec(memory_space=pltpu.MemorySpace.VMEM),
            pl.BlockSpec(memory_space=pltpu.MemorySpace.VMEM),
            pl.BlockSpec(memory_space=pltpu.MemorySpace.VMEM),
            pl.BlockSpec(memory_space=pl.ANY),
        ],
        out_specs=pl.BlockSpec(memory_space=pltpu.MemorySpace.VMEM),
        out_shape=jax.ShapeDtypeStruct((n, nclass), jnp.float32),
        scratch_shapes=[
            pltpu.VMEM((n, nclass), jnp.float32),
            pltpu.VMEM((_NBUF, _CH, n), jnp.float32),
            pltpu.SemaphoreType.DMA((_NBUF, 2)),
        ],
    )(x, W, b2, adj)
